# baseline (speedup 1.0000x reference)
import functools

import jax
import jax.numpy as jnp
from jax import lax
from jax.experimental import pallas as pl
from jax.experimental.pallas import tpu as pltpu
from jax.experimental.pallas import tpu_sc as plsc

E = 8
TOP_K = 2
NC = 2
NS = 16
NW = NC * NS
CH = 16


def _dot_nt(a, b):
    return jax.lax.dot_general(
        a, b, (((1,), (1,)), ((), ())), preferred_element_type=jnp.float32
    )


def _router_kernel(x_ref, rw_ref, o_ref):
    o_ref[...] = _dot_nt(x_ref[...], rw_ref[...])


def _shared_kernel(nj, x_ref, sg_ref, su_ref, sd_ref, sgw_ref, o_ref, logit_ref):
    j = pl.program_id(0)
    x = x_ref[...]

    @pl.when(j == 0)
    def _():
        prod = x.astype(jnp.float32) * sgw_ref[...]
        logit_ref[...] = jnp.sum(prod, axis=1, keepdims=True)
        o_ref[...] = jnp.zeros_like(o_ref)

    g = _dot_nt(x, sg_ref[...])
    u = _dot_nt(x, su_ref[...])
    act = ((g * jax.nn.sigmoid(g)) * u).astype(jnp.bfloat16)
    o_ref[...] += _dot_nt(act, sd_ref[...])

    @pl.when(j == nj - 1)
    def _():
        o_ref[...] = o_ref[...] * jax.nn.sigmoid(logit_ref[...])


def _grouped_kernel(emap_ref, xs_ref, gw_ref, uw_ref, dw_ref, w_ref, o_ref):
    x = xs_ref[...].astype(jnp.bfloat16)
    g = _dot_nt(x, gw_ref[0])
    u = _dot_nt(x, uw_ref[0])
    act = (g * jax.nn.sigmoid(g)) * u
    act = (act * w_ref[...]).astype(jnp.bfloat16)
    o_ref[...] = _dot_nt(act, dw_ref[0])


def _combine_kernel(a_ref, b_ref, c_ref, o_ref):
    o_ref[...] = a_ref[...] + b_ref[...] + c_ref[...]


def _sc_gather_body(nchunks, src_ref, idx_ref, out_ref, idx_all, r0, r1,
                    sg0, sg1, ss0, ss1):
    wid = lax.axis_index("s") * NC + lax.axis_index("c")
    base = wid * (nchunks * CH)
    rows = [r0, r1]
    gsem = [sg0, sg1]
    ssem = [ss0, ss1]

    pltpu.sync_copy(idx_ref.at[pl.ds(base, nchunks * CH)], idx_all)

    gather_h = [None, None]
    store_h = [None, None]
    gather_h[0] = pltpu.async_copy(
        src_ref.at[idx_all.at[pl.ds(0, CH)]], rows[0], gsem[0])
    for i in range(nchunks):
        cur, nxt = i % 2, (i + 1) % 2
        gather_h[cur].wait()
        if i + 1 < nchunks:
            if store_h[nxt] is not None:
                store_h[nxt].wait()
            gather_h[nxt] = pltpu.async_copy(
                src_ref.at[idx_all.at[pl.ds((i + 1) * CH, CH)]],
                rows[nxt], gsem[nxt])
        store_h[cur] = pltpu.async_copy(
            rows[cur], out_ref.at[pl.ds(base + i * CH, CH)], ssem[cur])
    for h in store_h:
        if h is not None:
            h.wait()


def _sc_gather(src, idx, n_rows, d):
    nchunks = n_rows // (NW * CH)
    mesh = plsc.VectorSubcoreMesh(core_axis_name="c", subcore_axis_name="s")
    fn = functools.partial(
        pl.kernel,
        mesh=mesh,
        out_type=jax.ShapeDtypeStruct((n_rows, d), jnp.float32),
        scratch_types=[
            pltpu.VMEM((nchunks * CH,), jnp.int32),
            pltpu.VMEM((CH, d), jnp.float32),
            pltpu.VMEM((CH, d), jnp.float32),
            pltpu.SemaphoreType.DMA,
            pltpu.SemaphoreType.DMA,
            pltpu.SemaphoreType.DMA,
            pltpu.SemaphoreType.DMA,
        ],
    )(functools.partial(_sc_gather_body, nchunks))
    return fn(src, idx)


def kernel(hidden_states, router_weight, gate_up_proj, down_proj,
           shared_gate_w, shared_up_w, shared_down_w, shared_expert_gate_w):
    b, s, h = hidden_states.shape
    x = hidden_states.reshape(-1, h)
    N = x.shape[0]
    I = down_proj.shape[2]
    SI = shared_gate_w.shape[0]
    BMG = 128
    P = N * TOP_K + E * BMG
    NB = P // BMG

    xb = x.astype(jnp.bfloat16)
    gw = gate_up_proj[:, :I].astype(jnp.bfloat16)
    uw = gate_up_proj[:, I:].astype(jnp.bfloat16)
    dw = down_proj.astype(jnp.bfloat16)
    sg = shared_gate_w.astype(jnp.bfloat16)
    su = shared_up_w.astype(jnp.bfloat16)
    sd = shared_down_w.astype(jnp.bfloat16)
    sgw = shared_expert_gate_w.astype(jnp.float32)

    router_logits = pl.pallas_call(
        _router_kernel,
        out_shape=jax.ShapeDtypeStruct((N, E), jnp.float32),
    )(xb, router_weight.astype(jnp.bfloat16))

    router_probs_full = jax.nn.softmax(router_logits, axis=-1)
    router_top_value, router_indices = jax.lax.top_k(router_probs_full, TOP_K)
    router_top_value = router_top_value / router_top_value.sum(axis=-1, keepdims=True)
    i0, i1 = router_indices[:, 0], router_indices[:, 1]
    w0, w1 = router_top_value[:, 0], router_top_value[:, 1]

    m = (jax.nn.one_hot(i0, E, dtype=jnp.int32)
         + jax.nn.one_hot(i1, E, dtype=jnp.int32))
    cum = jnp.cumsum(m, axis=0) - m
    counts = m.sum(axis=0)
    aligned = ((counts + BMG - 1) // BMG) * BMG
    ao = jnp.concatenate([jnp.zeros((1,), jnp.int32),
                          jnp.cumsum(aligned)]).astype(jnp.int32)
    r0 = jnp.take_along_axis(cum, i0[:, None], axis=1)[:, 0]
    r1 = jnp.take_along_axis(cum, i1[:, None], axis=1)[:, 0]
    q0 = ao[i0] + r0
    q1 = ao[i1] + r1
    tokens = jnp.arange(N, dtype=jnp.int32)
    tok_pad = jnp.zeros((P,), jnp.int32).at[q0].set(tokens).at[q1].set(tokens)
    w_pad = (jnp.zeros((P,), jnp.float32).at[q0].set(w0).at[q1].set(w1)
             .reshape(P, 1))
    emap = jnp.minimum(
        jnp.searchsorted(ao[1:], jnp.arange(NB, dtype=jnp.int32) * BMG,
                         side='right'),
        E - 1).astype(jnp.int32)
    qq = jnp.concatenate([q0, q1]).astype(jnp.int32)

    BS = 512
    NJ = SI // BS
    shared_gated = pl.pallas_call(
        functools.partial(_shared_kernel, NJ),
        grid=(NJ,),
        in_specs=[
            pl.BlockSpec((N, h), lambda j: (0, 0)),
            pl.BlockSpec((BS, h), lambda j: (j, 0)),
            pl.BlockSpec((BS, h), lambda j: (j, 0)),
            pl.BlockSpec((h, BS), lambda j: (0, j)),
            pl.BlockSpec((1, h), lambda j: (0, 0)),
        ],
        out_specs=pl.BlockSpec((N, h), lambda j: (0, 0)),
        out_shape=jax.ShapeDtypeStruct((N, h), jnp.float32),
        scratch_shapes=[pltpu.VMEM((N, 1), jnp.float32)],
        compiler_params=pltpu.CompilerParams(
            dimension_semantics=("arbitrary",),
        ),
    )(xb, sg, su, sd, sgw)

    xs = _sc_gather(x, tok_pad, P, h)

    grid_spec = pltpu.PrefetchScalarGridSpec(
        num_scalar_prefetch=1,
        grid=(NB,),
        in_specs=[
            pl.BlockSpec((BMG, h), lambda mi, emap_r: (mi, 0)),
            pl.BlockSpec((1, I, h), lambda mi, emap_r: (emap_r[mi], 0, 0)),
            pl.BlockSpec((1, I, h), lambda mi, emap_r: (emap_r[mi], 0, 0)),
            pl.BlockSpec((1, h, I), lambda mi, emap_r: (emap_r[mi], 0, 0)),
            pl.BlockSpec((BMG, 1), lambda mi, emap_r: (mi, 0)),
        ],
        out_specs=pl.BlockSpec((BMG, h), lambda mi, emap_r: (mi, 0)),
    )
    ys = pl.pallas_call(
        _grouped_kernel,
        grid_spec=grid_spec,
        out_shape=jax.ShapeDtypeStruct((P, h), jnp.float32),
        compiler_params=pltpu.CompilerParams(
            dimension_semantics=("arbitrary",),
        ),
    )(emap, xs, gw, uw, dw, w_pad)

    yy = _sc_gather(ys, qq, TOP_K * N, h)

    BC = 512
    NBLK = N // BC
    expert_output = pl.pallas_call(
        _combine_kernel,
        grid=(NBLK,),
        in_specs=[
            pl.BlockSpec((BC, h), lambda i: (i, 0)),
            pl.BlockSpec((BC, h), lambda i: (NBLK + i, 0)),
            pl.BlockSpec((BC, h), lambda i: (i, 0)),
        ],
        out_specs=pl.BlockSpec((BC, h), lambda i: (i, 0)),
        out_shape=jax.ShapeDtypeStruct((N, h), jnp.float32),
        compiler_params=pltpu.CompilerParams(
            dimension_semantics=("arbitrary",),
        ),
    )(yy, yy, shared_gated)

    expert_mask = jax.nn.one_hot(router_indices, E, dtype=jnp.float32)
    tokens_per_expert = expert_mask.sum(axis=(0, 1))
    fraction_tokens = tokens_per_expert / (N * TOP_K)
    router_probs_summed = jax.nn.softmax(router_logits, axis=-1).sum(axis=0)
    fraction_probs = router_probs_summed.sum() / N
    aux_loss = E * jnp.sum(fraction_tokens * fraction_probs)

    return (expert_output.reshape(b, s, h), aux_loss)


# speedup vs baseline: 1.1023x; 1.1023x over previous
import functools

import jax
import jax.numpy as jnp
from jax import lax
from jax.experimental import pallas as pl
from jax.experimental.pallas import tpu as pltpu
from jax.experimental.pallas import tpu_sc as plsc

E = 8
TOP_K = 2
NC = 2
NS = 16
NW = NC * NS
CH = 16


def _dot_nt(a, b):
    return jax.lax.dot_general(
        a, b, (((1,), (1,)), ((), ())), preferred_element_type=jnp.float32
    )


def _router_kernel(x_ref, rw_ref, o_ref):
    o_ref[...] = _dot_nt(x_ref[...], rw_ref[...])


def _shared_kernel(nj, x_ref, sg_ref, su_ref, sd_ref, sgw_ref, o_ref, logit_ref):
    j = pl.program_id(0)
    x = x_ref[...]

    @pl.when(j == 0)
    def _():
        prod = x.astype(jnp.float32) * sgw_ref[...]
        logit_ref[...] = jnp.sum(prod, axis=1, keepdims=True)
        o_ref[...] = jnp.zeros_like(o_ref)

    g = _dot_nt(x, sg_ref[...])
    u = _dot_nt(x, su_ref[...])
    act = ((g * jax.nn.sigmoid(g)) * u).astype(jnp.bfloat16)
    o_ref[...] += _dot_nt(act, sd_ref[...])

    @pl.when(j == nj - 1)
    def _():
        o_ref[...] = o_ref[...] * jax.nn.sigmoid(logit_ref[...])


def _grouped_kernel(emap_ref, xs_ref, gw_ref, uw_ref, dw_ref, w_ref, o_ref):
    x = xs_ref[...].astype(jnp.bfloat16)
    g = _dot_nt(x, gw_ref[0])
    u = _dot_nt(x, uw_ref[0])
    act = (g * jax.nn.sigmoid(g)) * u
    act = (act * w_ref[...]).astype(jnp.bfloat16)
    o_ref[...] = _dot_nt(act, dw_ref[0])


def _combine_kernel(a_ref, b_ref, c_ref, o_ref):
    o_ref[...] = a_ref[...] + b_ref[...] + c_ref[...]


def _sc_gather_body(nchunks, src_ref, idx_ref, out_ref, idx_all, r0, r1,
                    sg0, sg1, ss0, ss1):
    wid = lax.axis_index("s") * NC + lax.axis_index("c")
    base = wid * (nchunks * CH)
    rows = [r0, r1]
    gsem = [sg0, sg1]
    ssem = [ss0, ss1]

    pltpu.sync_copy(idx_ref.at[pl.ds(base, nchunks * CH)], idx_all)

    gather_h = [None, None]
    store_h = [None, None]
    gather_h[0] = pltpu.async_copy(
        src_ref.at[idx_all.at[pl.ds(0, CH)]], rows[0], gsem[0])
    for i in range(nchunks):
        cur, nxt = i % 2, (i + 1) % 2
        gather_h[cur].wait()
        if i + 1 < nchunks:
            if store_h[nxt] is not None:
                store_h[nxt].wait()
            gather_h[nxt] = pltpu.async_copy(
                src_ref.at[idx_all.at[pl.ds((i + 1) * CH, CH)]],
                rows[nxt], gsem[nxt])
        store_h[cur] = pltpu.async_copy(
            rows[cur], out_ref.at[pl.ds(base + i * CH, CH)], ssem[cur])
    for h in store_h:
        if h is not None:
            h.wait()


def _sc_gather(src, idx, n_rows, d):
    nchunks = n_rows // (NW * CH)
    mesh = plsc.VectorSubcoreMesh(core_axis_name="c", subcore_axis_name="s")
    fn = functools.partial(
        pl.kernel,
        mesh=mesh,
        out_type=jax.ShapeDtypeStruct((n_rows, d), jnp.float32),
        scratch_types=[
            pltpu.VMEM((nchunks * CH,), jnp.int32),
            pltpu.VMEM((CH, d), jnp.float32),
            pltpu.VMEM((CH, d), jnp.float32),
            pltpu.SemaphoreType.DMA,
            pltpu.SemaphoreType.DMA,
            pltpu.SemaphoreType.DMA,
            pltpu.SemaphoreType.DMA,
        ],
    )(functools.partial(_sc_gather_body, nchunks))
    return fn(src, idx)


def kernel(hidden_states, router_weight, gate_up_proj, down_proj,
           shared_gate_w, shared_up_w, shared_down_w, shared_expert_gate_w):
    b, s, h = hidden_states.shape
    x = hidden_states.reshape(-1, h)
    N = x.shape[0]
    I = down_proj.shape[2]
    SI = shared_gate_w.shape[0]
    BMG = 128
    P = N * TOP_K + E * BMG
    NB = P // BMG

    xb = x.astype(jnp.bfloat16)
    gw = gate_up_proj[:, :I].astype(jnp.bfloat16)
    uw = gate_up_proj[:, I:].astype(jnp.bfloat16)
    dw = down_proj.astype(jnp.bfloat16)
    sg = shared_gate_w.astype(jnp.bfloat16)
    su = shared_up_w.astype(jnp.bfloat16)
    sd = shared_down_w.astype(jnp.bfloat16)
    sgw = shared_expert_gate_w.astype(jnp.float32)

    router_logits = pl.pallas_call(
        _router_kernel,
        out_shape=jax.ShapeDtypeStruct((N, E), jnp.float32),
    )(xb, router_weight.astype(jnp.bfloat16))

    router_probs_full = jax.nn.softmax(router_logits, axis=-1)
    router_top_value, router_indices = jax.lax.top_k(router_probs_full, TOP_K)
    router_top_value = router_top_value / router_top_value.sum(axis=-1, keepdims=True)
    i0, i1 = router_indices[:, 0], router_indices[:, 1]
    w0, w1 = router_top_value[:, 0], router_top_value[:, 1]

    m = (jax.nn.one_hot(i0, E, dtype=jnp.int32)
         + jax.nn.one_hot(i1, E, dtype=jnp.int32))
    cum = jnp.cumsum(m, axis=0) - m
    counts = m.sum(axis=0)
    aligned = ((counts + BMG - 1) // BMG) * BMG
    ao = jnp.concatenate([jnp.zeros((1,), jnp.int32),
                          jnp.cumsum(aligned)]).astype(jnp.int32)
    r0 = jnp.take_along_axis(cum, i0[:, None], axis=1)[:, 0]
    r1 = jnp.take_along_axis(cum, i1[:, None], axis=1)[:, 0]
    q0 = ao[i0] + r0
    q1 = ao[i1] + r1
    tokens = jnp.arange(N, dtype=jnp.int32)
    tok_pad = (jnp.arange(P, dtype=jnp.int32) % N).at[q0].set(tokens).at[q1].set(tokens)
    w_pad = (jnp.zeros((P,), jnp.float32).at[q0].set(w0).at[q1].set(w1)
             .reshape(P, 1))
    emap = jnp.minimum(
        jnp.searchsorted(ao[1:], jnp.arange(NB, dtype=jnp.int32) * BMG,
                         side='right'),
        E - 1).astype(jnp.int32)
    qq = jnp.concatenate([q0, q1]).astype(jnp.int32)

    BS = 512
    NJ = SI // BS
    shared_gated = pl.pallas_call(
        functools.partial(_shared_kernel, NJ),
        grid=(NJ,),
        in_specs=[
            pl.BlockSpec((N, h), lambda j: (0, 0)),
            pl.BlockSpec((BS, h), lambda j: (j, 0)),
            pl.BlockSpec((BS, h), lambda j: (j, 0)),
            pl.BlockSpec((h, BS), lambda j: (0, j)),
            pl.BlockSpec((1, h), lambda j: (0, 0)),
        ],
        out_specs=pl.BlockSpec((N, h), lambda j: (0, 0)),
        out_shape=jax.ShapeDtypeStruct((N, h), jnp.float32),
        scratch_shapes=[pltpu.VMEM((N, 1), jnp.float32)],
        compiler_params=pltpu.CompilerParams(
            dimension_semantics=("arbitrary",),
        ),
    )(xb, sg, su, sd, sgw)

    xs = _sc_gather(x, tok_pad, P, h)

    grid_spec = pltpu.PrefetchScalarGridSpec(
        num_scalar_prefetch=1,
        grid=(NB,),
        in_specs=[
            pl.BlockSpec((BMG, h), lambda mi, emap_r: (mi, 0)),
            pl.BlockSpec((1, I, h), lambda mi, emap_r: (emap_r[mi], 0, 0)),
            pl.BlockSpec((1, I, h), lambda mi, emap_r: (emap_r[mi], 0, 0)),
            pl.BlockSpec((1, h, I), lambda mi, emap_r: (emap_r[mi], 0, 0)),
            pl.BlockSpec((BMG, 1), lambda mi, emap_r: (mi, 0)),
        ],
        out_specs=pl.BlockSpec((BMG, h), lambda mi, emap_r: (mi, 0)),
    )
    ys = pl.pallas_call(
        _grouped_kernel,
        grid_spec=grid_spec,
        out_shape=jax.ShapeDtypeStruct((P, h), jnp.float32),
        compiler_params=pltpu.CompilerParams(
            dimension_semantics=("arbitrary",),
        ),
    )(emap, xs, gw, uw, dw, w_pad)

    yy = _sc_gather(ys, qq, TOP_K * N, h)

    BC = 512
    NBLK = N // BC
    expert_output = pl.pallas_call(
        _combine_kernel,
        grid=(NBLK,),
        in_specs=[
            pl.BlockSpec((BC, h), lambda i: (i, 0)),
            pl.BlockSpec((BC, h), lambda i: (NBLK + i, 0)),
            pl.BlockSpec((BC, h), lambda i: (i, 0)),
        ],
        out_specs=pl.BlockSpec((BC, h), lambda i: (i, 0)),
        out_shape=jax.ShapeDtypeStruct((N, h), jnp.float32),
        compiler_params=pltpu.CompilerParams(
            dimension_semantics=("arbitrary",),
        ),
    )(yy, yy, shared_gated)

    expert_mask = jax.nn.one_hot(router_indices, E, dtype=jnp.float32)
    tokens_per_expert = expert_mask.sum(axis=(0, 1))
    fraction_tokens = tokens_per_expert / (N * TOP_K)
    router_probs_summed = jax.nn.softmax(router_logits, axis=-1).sum(axis=0)
    fraction_probs = router_probs_summed.sum() / N
    aux_loss = E * jnp.sum(fraction_tokens * fraction_probs)

    return (expert_output.reshape(b, s, h), aux_loss)


# speedup vs baseline: 1.3002x; 1.1796x over previous
import functools

import jax
import jax.numpy as jnp
from jax import lax
from jax.experimental import pallas as pl
from jax.experimental.pallas import tpu as pltpu
from jax.experimental.pallas import tpu_sc as plsc

E = 8
TOP_K = 2
NC = 2
NS = 16
NW = NC * NS
CH = 16


def _dot_nt(a, b):
    return jax.lax.dot_general(
        a, b, (((1,), (1,)), ((), ())), preferred_element_type=jnp.float32
    )


def _router_kernel(x_ref, rw_ref, o_ref):
    o_ref[...] = _dot_nt(x_ref[...], rw_ref[...])


def _shared_kernel(nj, x_ref, sg_ref, su_ref, sd_ref, sgw_ref, o_ref, logit_ref):
    j = pl.program_id(0)
    x = x_ref[...]

    @pl.when(j == 0)
    def _():
        prod = x.astype(jnp.float32) * sgw_ref[...]
        logit_ref[...] = jnp.sum(prod, axis=1, keepdims=True)
        o_ref[...] = jnp.zeros_like(o_ref)

    g = _dot_nt(x, sg_ref[...])
    u = _dot_nt(x, su_ref[...])
    act = ((g * jax.nn.sigmoid(g)) * u).astype(jnp.bfloat16)
    o_ref[...] += _dot_nt(act, sd_ref[...])

    @pl.when(j == nj - 1)
    def _():
        o_ref[...] = o_ref[...] * jax.nn.sigmoid(logit_ref[...])


def _grouped_kernel(emap_ref, xs_ref, gw_ref, uw_ref, dw_ref, w_ref, o_ref):
    x = xs_ref[...].astype(jnp.bfloat16)
    g = _dot_nt(x, gw_ref[0])
    u = _dot_nt(x, uw_ref[0])
    act = (g * jax.nn.sigmoid(g)) * u
    act = (act * w_ref[...]).astype(jnp.bfloat16)
    o_ref[...] = _dot_nt(act, dw_ref[0])


def _combine_kernel(a_ref, b_ref, c_ref, o_ref):
    o_ref[...] = a_ref[...] + b_ref[...] + c_ref[...]


def _sc_gather_body(nchunks, src_ref, idx_ref, out_ref, idx_all, r0, r1,
                    sg0, sg1, ss0, ss1):
    wid = lax.axis_index("s") * NC + lax.axis_index("c")
    base = wid * (nchunks * CH)
    rows = [r0, r1]
    gsem = [sg0, sg1]
    ssem = [ss0, ss1]

    pltpu.sync_copy(idx_ref.at[pl.ds(base, nchunks * CH)], idx_all)

    gather_h = [None, None]
    store_h = [None, None]
    gather_h[0] = pltpu.async_copy(
        src_ref.at[idx_all.at[pl.ds(0, CH)]], rows[0], gsem[0])
    for i in range(nchunks):
        cur, nxt = i % 2, (i + 1) % 2
        gather_h[cur].wait()
        if i + 1 < nchunks:
            if store_h[nxt] is not None:
                store_h[nxt].wait()
            gather_h[nxt] = pltpu.async_copy(
                src_ref.at[idx_all.at[pl.ds((i + 1) * CH, CH)]],
                rows[nxt], gsem[nxt])
        store_h[cur] = pltpu.async_copy(
            rows[cur], out_ref.at[pl.ds(base + i * CH, CH)], ssem[cur])
    for h in store_h:
        if h is not None:
            h.wait()


def _sc_gather(src, idx, n_rows, d):
    nchunks = n_rows // (NW * CH)
    mesh = plsc.VectorSubcoreMesh(core_axis_name="c", subcore_axis_name="s")
    fn = functools.partial(
        pl.kernel,
        mesh=mesh,
        out_type=jax.ShapeDtypeStruct((n_rows, d), jnp.float32),
        scratch_types=[
            pltpu.VMEM((nchunks * CH,), jnp.int32),
            pltpu.VMEM((CH, d), jnp.float32),
            pltpu.VMEM((CH, d), jnp.float32),
            pltpu.SemaphoreType.DMA,
            pltpu.SemaphoreType.DMA,
            pltpu.SemaphoreType.DMA,
            pltpu.SemaphoreType.DMA,
        ],
    )(functools.partial(_sc_gather_body, nchunks))
    return fn(src, idx)


def kernel(hidden_states, router_weight, gate_up_proj, down_proj,
           shared_gate_w, shared_up_w, shared_down_w, shared_expert_gate_w):
    b, s, h = hidden_states.shape
    x = hidden_states.reshape(-1, h)
    N = x.shape[0]
    I = down_proj.shape[2]
    SI = shared_gate_w.shape[0]
    BMG = 256
    P = N * TOP_K + E * BMG
    NB = P // BMG

    xb = x.astype(jnp.bfloat16)
    gw = gate_up_proj[:, :I].astype(jnp.bfloat16)
    uw = gate_up_proj[:, I:].astype(jnp.bfloat16)
    dw = down_proj.astype(jnp.bfloat16)
    sg = shared_gate_w.astype(jnp.bfloat16)
    su = shared_up_w.astype(jnp.bfloat16)
    sd = shared_down_w.astype(jnp.bfloat16)
    sgw = shared_expert_gate_w.astype(jnp.float32)

    router_logits = pl.pallas_call(
        _router_kernel,
        out_shape=jax.ShapeDtypeStruct((N, E), jnp.float32),
    )(xb, router_weight.astype(jnp.bfloat16))

    router_probs_full = jax.nn.softmax(router_logits, axis=-1)
    router_top_value, router_indices = jax.lax.top_k(router_probs_full, TOP_K)
    router_top_value = router_top_value / router_top_value.sum(axis=-1, keepdims=True)
    i0, i1 = router_indices[:, 0], router_indices[:, 1]
    w0, w1 = router_top_value[:, 0], router_top_value[:, 1]

    m = (jax.nn.one_hot(i0, E, dtype=jnp.int32)
         + jax.nn.one_hot(i1, E, dtype=jnp.int32))
    cum = jnp.cumsum(m, axis=0) - m
    counts = m.sum(axis=0)
    aligned = ((counts + BMG - 1) // BMG) * BMG
    ao = jnp.concatenate([jnp.zeros((1,), jnp.int32),
                          jnp.cumsum(aligned)]).astype(jnp.int32)
    r0 = jnp.take_along_axis(cum, i0[:, None], axis=1)[:, 0]
    r1 = jnp.take_along_axis(cum, i1[:, None], axis=1)[:, 0]
    q0 = ao[i0] + r0
    q1 = ao[i1] + r1
    tokens = jnp.arange(N, dtype=jnp.int32)
    tok_pad = (jnp.arange(P, dtype=jnp.int32) % N).at[q0].set(tokens).at[q1].set(tokens)
    w_pad = (jnp.zeros((P,), jnp.float32).at[q0].set(w0).at[q1].set(w1)
             .reshape(P, 1))
    emap = jnp.minimum(
        jnp.searchsorted(ao[1:], jnp.arange(NB, dtype=jnp.int32) * BMG,
                         side='right'),
        E - 1).astype(jnp.int32)
    qq = jnp.concatenate([q0, q1]).astype(jnp.int32)

    BS = 512
    NJ = SI // BS
    shared_gated = pl.pallas_call(
        functools.partial(_shared_kernel, NJ),
        grid=(NJ,),
        in_specs=[
            pl.BlockSpec((N, h), lambda j: (0, 0)),
            pl.BlockSpec((BS, h), lambda j: (j, 0)),
            pl.BlockSpec((BS, h), lambda j: (j, 0)),
            pl.BlockSpec((h, BS), lambda j: (0, j)),
            pl.BlockSpec((1, h), lambda j: (0, 0)),
        ],
        out_specs=pl.BlockSpec((N, h), lambda j: (0, 0)),
        out_shape=jax.ShapeDtypeStruct((N, h), jnp.float32),
        scratch_shapes=[pltpu.VMEM((N, 1), jnp.float32)],
        compiler_params=pltpu.CompilerParams(
            dimension_semantics=("arbitrary",),
        ),
    )(xb, sg, su, sd, sgw)

    xs = _sc_gather(x, tok_pad, P, h)

    grid_spec = pltpu.PrefetchScalarGridSpec(
        num_scalar_prefetch=1,
        grid=(NB,),
        in_specs=[
            pl.BlockSpec((BMG, h), lambda mi, emap_r: (mi, 0)),
            pl.BlockSpec((1, I, h), lambda mi, emap_r: (emap_r[mi], 0, 0)),
            pl.BlockSpec((1, I, h), lambda mi, emap_r: (emap_r[mi], 0, 0)),
            pl.BlockSpec((1, h, I), lambda mi, emap_r: (emap_r[mi], 0, 0)),
            pl.BlockSpec((BMG, 1), lambda mi, emap_r: (mi, 0)),
        ],
        out_specs=pl.BlockSpec((BMG, h), lambda mi, emap_r: (mi, 0)),
    )
    ys = pl.pallas_call(
        _grouped_kernel,
        grid_spec=grid_spec,
        out_shape=jax.ShapeDtypeStruct((P, h), jnp.float32),
        compiler_params=pltpu.CompilerParams(
            dimension_semantics=("arbitrary",),
        ),
    )(emap, xs, gw, uw, dw, w_pad)

    yy = _sc_gather(ys, qq, TOP_K * N, h)

    BC = 512
    NBLK = N // BC
    expert_output = pl.pallas_call(
        _combine_kernel,
        grid=(NBLK,),
        in_specs=[
            pl.BlockSpec((BC, h), lambda i: (i, 0)),
            pl.BlockSpec((BC, h), lambda i: (NBLK + i, 0)),
            pl.BlockSpec((BC, h), lambda i: (i, 0)),
        ],
        out_specs=pl.BlockSpec((BC, h), lambda i: (i, 0)),
        out_shape=jax.ShapeDtypeStruct((N, h), jnp.float32),
        compiler_params=pltpu.CompilerParams(
            dimension_semantics=("arbitrary",),
        ),
    )(yy, yy, shared_gated)

    expert_mask = jax.nn.one_hot(router_indices, E, dtype=jnp.float32)
    tokens_per_expert = expert_mask.sum(axis=(0, 1))
    fraction_tokens = tokens_per_expert / (N * TOP_K)
    router_probs_summed = jax.nn.softmax(router_logits, axis=-1).sum(axis=0)
    fraction_probs = router_probs_summed.sum() / N
    aux_loss = E * jnp.sum(fraction_tokens * fraction_probs)

    return (expert_output.reshape(b, s, h), aux_loss)
